# Initial kernel scaffold; baseline (speedup 1.0000x reference)
import jax
import jax.numpy as jnp
from jax.experimental import pallas as pl


def kernel(x, table):
    raise NotImplementedError("write your pallas kernel here")


# speedup vs baseline: 3.5822x; 3.5822x over previous
import functools

import jax
import jax.numpy as jnp
from jax import lax
from jax.experimental import pallas as pl
from jax.experimental.pallas import tpu as pltpu
from jax.experimental.pallas import tpu_sc as plsc

_DELTA = 10
_ACTIONS = 512
_EMB = 32
_PLANES = 2 * _DELTA + 1

_NC = 2
_NS = 16
_NW = _NC * _NS
_AG = _ACTIONS // _NW
_NB = 64


def _spline_body(x_hbm, tbl_hbm, out_hbm, tbl_v, x_v, out_v):
    wid = lax.axis_index("s") * _NC + lax.axis_index("c")
    a0 = wid * _AG
    n_total = x_hbm.shape[0]

    pltpu.sync_copy(tbl_hbm.at[:, pl.ds(a0, _AG), :], tbl_v)

    iota = lax.iota(jnp.int32, _AG)
    inv_d = jnp.float32(1.0 / _DELTA)

    @pl.loop(0, n_total // _NB)
    def _block(b):
        n0 = b * _NB
        pltpu.sync_copy(x_hbm.at[pl.ds(n0, _NB), pl.ds(a0, _AG)], x_v)

        @pl.loop(0, _NB)
        def _row(i):
            iv = jnp.full((_AG,), i, jnp.int32)
            xi = plsc.load_gather(x_v, [iv, iota])
            t = xi * jnp.float32(_DELTA)
            xl_i = t.astype(jnp.int32)
            xh_i = (t + 1.0).astype(jnp.float32).astype(jnp.int32)
            xl_s = xl_i.astype(jnp.float32) / jnp.float32(_DELTA)
            xh_s = xh_i.astype(jnp.float32) / jnp.float32(_DELTA)
            w_h = (xi - xl_s) / inv_d
            w_l = (xh_s - xi) / inv_d
            plane_l = xl_i + _DELTA
            plane_h = jnp.minimum(xh_i + _DELTA, _PLANES - 1)

            for e in range(_EMB):
                ev = jnp.full((_AG,), e, jnp.int32)
                bl = plsc.load_gather(tbl_v, [plane_l, iota, ev])
                bh = plsc.load_gather(tbl_v, [plane_h, iota, ev])
                val = bh * w_h + bl * w_l
                plsc.store_scatter(out_v, [iv, iota, ev], val)

        pltpu.sync_copy(out_v, out_hbm.at[pl.ds(n0, _NB), pl.ds(a0, _AG), :])


@jax.jit
def kernel(x, table):
    n = x.shape[0]
    tbl3 = table.reshape(_PLANES, _ACTIONS, _EMB)
    mesh = plsc.VectorSubcoreMesh(core_axis_name="c", subcore_axis_name="s")
    run = pl.kernel(
        _spline_body,
        out_type=jax.ShapeDtypeStruct((n, _ACTIONS, _EMB), jnp.float32),
        mesh=mesh,
        scratch_types=[
            pltpu.VMEM((_PLANES, _AG, _EMB), jnp.float32),
            pltpu.VMEM((_NB, _AG), jnp.float32),
            pltpu.VMEM((_NB, _AG, _EMB), jnp.float32),
        ],
        compiler_params=pltpu.CompilerParams(
            use_tc_tiling_on_sc=False, needs_layout_passes=False
        ),
    )
    return run(x, tbl3)


# speedup vs baseline: 7.2387x; 2.0207x over previous
import functools

import jax
import jax.numpy as jnp
from jax import lax
from jax.experimental import pallas as pl
from jax.experimental.pallas import tpu as pltpu
from jax.experimental.pallas import tpu_sc as plsc

_DELTA = 10
_ACTIONS = 512
_EMB = 32
_PLANES = 2 * _DELTA + 1

_NC = 2
_NS = 16
_NW = _NC * _NS
_AG = _ACTIONS // _NW
_NB = 64


_PAD = _EMB + 1


def _spline_body(x_hbm, tbl_hbm, out_hbm, tbl_v, x_v, out_v):
    wid = lax.axis_index("s") * _NC + lax.axis_index("c")
    a0 = wid * _AG
    n_total = x_hbm.shape[0]

    pltpu.sync_copy(tbl_hbm.at[:, pl.ds(a0, _AG), :], tbl_v.at[:, :, pl.ds(0, _EMB)])

    iota = lax.iota(jnp.int32, _AG)
    inv_d = jnp.float32(1.0 / _DELTA)

    @pl.loop(0, n_total // _NB)
    def _block(b):
        n0 = b * _NB
        pltpu.sync_copy(x_hbm.at[pl.ds(n0, _NB), pl.ds(a0, _AG)], x_v)

        @pl.loop(0, _NB)
        def _row(i):
            iv = jnp.full((_AG,), i, jnp.int32)
            xi = plsc.load_gather(x_v, [iv, iota])
            t = xi * jnp.float32(_DELTA)
            xl_i = t.astype(jnp.int32)
            xh_i = (t + 1.0).astype(jnp.float32).astype(jnp.int32)
            xl_s = xl_i.astype(jnp.float32) / jnp.float32(_DELTA)
            xh_s = xh_i.astype(jnp.float32) / jnp.float32(_DELTA)
            w_h = (xi - xl_s) / inv_d
            w_l = (xh_s - xi) / inv_d
            plane_l = xl_i + _DELTA
            plane_h = jnp.minimum(xh_i + _DELTA, _PLANES - 1)

            for e in range(_EMB):
                ev = jnp.full((_AG,), e, jnp.int32)
                bl = plsc.load_gather(tbl_v, [plane_l, iota, ev])
                bh = plsc.load_gather(tbl_v, [plane_h, iota, ev])
                val = bh * w_h + bl * w_l
                plsc.store_scatter(out_v, [iv, iota, ev], val)

        pltpu.sync_copy(
            out_v.at[:, :, pl.ds(0, _EMB)],
            out_hbm.at[pl.ds(n0, _NB), pl.ds(a0, _AG), :],
        )


@jax.jit
def kernel(x, table):
    n = x.shape[0]
    tbl3 = table.reshape(_PLANES, _ACTIONS, _EMB)
    mesh = plsc.VectorSubcoreMesh(core_axis_name="c", subcore_axis_name="s")
    run = pl.kernel(
        _spline_body,
        out_type=jax.ShapeDtypeStruct((n, _ACTIONS, _EMB), jnp.float32),
        mesh=mesh,
        scratch_types=[
            pltpu.VMEM((_PLANES, _AG, _PAD), jnp.float32),
            pltpu.VMEM((_NB, _AG), jnp.float32),
            pltpu.VMEM((_NB, _AG, _PAD), jnp.float32),
        ],
        compiler_params=pltpu.CompilerParams(
            use_tc_tiling_on_sc=False, needs_layout_passes=False
        ),
    )
    return run(x, tbl3)


# speedup vs baseline: 11.2089x; 1.5485x over previous
import functools

import jax
import jax.numpy as jnp
from jax import lax
from jax.experimental import pallas as pl
from jax.experimental.pallas import tpu as pltpu
from jax.experimental.pallas import tpu_sc as plsc

_DELTA = 10
_ACTIONS = 512
_EMB = 32
_PLANES = 2 * _DELTA + 1

_NC = 2
_NS = 16
_NW = _NC * _NS
_AG = _ACTIONS // _NW
_NB = 64


_PAD = _EMB + 1


def _spline_body(x_hbm, tbl_hbm, out_hbm, tbl_v, x_v, out_v):
    wid = lax.axis_index("s") * _NC + lax.axis_index("c")
    a0 = wid * _AG
    n_total = x_hbm.shape[0]

    pltpu.sync_copy(tbl_hbm.at[:, pl.ds(a0, _AG), :], tbl_v.at[:, :, pl.ds(0, _EMB)])

    iota = lax.iota(jnp.int32, _AG)
    inv_d = jnp.float32(1.0 / _DELTA)

    @pl.loop(0, n_total // _NB)
    def _block(b):
        n0 = b * _NB
        pltpu.sync_copy(x_hbm.at[pl.ds(n0, _NB), pl.ds(a0, _AG)], x_v)

        @plsc.parallel_loop(0, _NB, unroll=2)
        def _row(i):
            iv = jnp.full((_AG,), i, jnp.int32)
            xi = plsc.load_gather(x_v, [iv, iota])
            t = xi * jnp.float32(_DELTA)
            xl_i = t.astype(jnp.int32)
            xh_i = (t + 1.0).astype(jnp.float32).astype(jnp.int32)
            xl_s = xl_i.astype(jnp.float32) / jnp.float32(_DELTA)
            xh_s = xh_i.astype(jnp.float32) / jnp.float32(_DELTA)
            w_h = (xi - xl_s) / inv_d
            w_l = (xh_s - xi) / inv_d
            plane_l = xl_i + _DELTA
            plane_h = jnp.minimum(xh_i + _DELTA, _PLANES - 1)

            for e in range(_EMB):
                ev = jnp.full((_AG,), e, jnp.int32)
                bl = plsc.load_gather(tbl_v, [plane_l, iota, ev])
                bh = plsc.load_gather(tbl_v, [plane_h, iota, ev])
                val = bh * w_h + bl * w_l
                plsc.store_scatter(out_v, [iv, iota, ev], val)

        pltpu.sync_copy(
            out_v.at[:, :, pl.ds(0, _EMB)],
            out_hbm.at[pl.ds(n0, _NB), pl.ds(a0, _AG), :],
        )


@jax.jit
def kernel(x, table):
    n = x.shape[0]
    tbl3 = table.reshape(_PLANES, _ACTIONS, _EMB)
    mesh = plsc.VectorSubcoreMesh(core_axis_name="c", subcore_axis_name="s")
    run = pl.kernel(
        _spline_body,
        out_type=jax.ShapeDtypeStruct((n, _ACTIONS, _EMB), jnp.float32),
        mesh=mesh,
        scratch_types=[
            pltpu.VMEM((_PLANES, _AG, _PAD), jnp.float32),
            pltpu.VMEM((_NB, _AG), jnp.float32),
            pltpu.VMEM((_NB, _AG, _PAD), jnp.float32),
        ],
        compiler_params=pltpu.CompilerParams(
            use_tc_tiling_on_sc=False, needs_layout_passes=False
        ),
    )
    return run(x, tbl3)
